# Initial kernel scaffold; baseline (speedup 1.0000x reference)
import jax
import jax.numpy as jnp
from jax.experimental import pallas as pl


def kernel(x, W_enc, b_enc, W_dec, W_bw, b_bw, neighbour_distance, neighbour_id, clustering_labels):
    raise NotImplementedError("write your pallas kernel here")


# speedup vs baseline: 8.2582x; 8.2582x over previous
import functools

import jax
import jax.numpy as jnp
from jax import lax
from jax.experimental import pallas as pl
from jax.experimental.pallas import tpu as pltpu
from jax.experimental.pallas import tpu_sc as plsc

_N = 50000
_NLAT = 16
_M = 16
_MU = 10.0
_K = 16
_B = 4
_LO = 2.0 / 60.0 / _MU
_HI = 1.0 / 60.0

_NW = 32
_PW = 1568
_CH = 16
_IDX = _CH * _K


def _tc_body(x_ref, we_ref, be_ref, wbwp_ref, bbwp_ref, enc_ref, inv2_ref):
    x = x_ref[...]
    we = we_ref[...]
    enc = lax.dot_general(
        x, we, (((1,), (1,)), ((), ())),
        preferred_element_type=jnp.float32,
        precision=lax.Precision.HIGHEST,
    ) + be_ref[...]
    enc_ref[...] = enc
    z = lax.dot_general(
        enc, wbwp_ref[...], (((1,), (1,)), ((), ())),
        preferred_element_type=jnp.float32,
        precision=lax.Precision.HIGHEST,
    ) + bbwp_ref[...]
    bw = (_HI - _LO) * jax.nn.sigmoid(z) + _LO
    inv2_ref[...] = 1.0 / (bw * _MU) ** 2


def _sc_body(wd_hbm, nid_hbm, d_hbm, lab_hbm, enc_hbm, inv2_hbm,
             out0, out1, out2, out3,
             idx_v, rows_v, d_v, lab_v, enc_v, inv2_v, zsc_v,
             ob0, ob1, ob2, ob3, sem):
    obufs = (ob0, ob1, ob2, ob3)
    outs = (out0, out1, out2, out3)
    w = lax.axis_index("s") * 2 + lax.axis_index("c")
    start = w * _PW
    nloc = jnp.where(w == _NW - 1, _N - (_NW - 1) * _PW, _PW)
    nch = nloc // _CH

    pltpu.sync_copy(enc_hbm, enc_v)
    pltpu.sync_copy(inv2_hbm, inv2_v)
    zidx = lax.iota(jnp.int32, 16) * 16

    def chunk_body(ch, carry):
        base = start + ch * _CH
        fo = base * _K
        pltpu.sync_copy(nid_hbm.at[pl.ds(fo, _IDX)], idx_v)
        pltpu.sync_copy(d_hbm.at[pl.ds(fo, _IDX)], d_v)
        pltpu.sync_copy(lab_hbm.at[pl.ds(base, _CH)], lab_v)
        cp0 = pltpu.async_copy(
            wd_hbm.at[idx_v.at[pl.ds(0, 128)]], rows_v.at[pl.ds(0, 128)], sem)
        cp1 = pltpu.async_copy(
            wd_hbm.at[idx_v.at[pl.ds(128, 128)]], rows_v.at[pl.ds(128, 128)], sem)
        cp0.wait()
        cp1.wait()
        labv = lab_v[...]
        for j in range(_CH):
            dv = d_v[pl.ds(j * _K, _K)]
            d2v = dv * dv
            s2 = jnp.float32(0.0)
            sg = jnp.zeros((16,), jnp.float32)
            sdg = jnp.zeros((16,), jnp.float32)
            for k in range(_K):
                g = rows_v[j * _K + k]
                d2k = d2v[k]
                s2 = s2 + d2k
                sg = sg + g
                sdg = sdg + jnp.full((16,), d2k) * g
            c = labv[j]
            s2v = jnp.full((16,), s2)
            for b in range(_B):
                iv = inv2_v[pl.ds(b * 256 + c * 16, 16)]
                num = sg - iv * sdg
                den = float(_K) - iv * s2v
                zb = enc_v[pl.ds(b * 16, 16)] * (num / den)
                plsc.store_scatter(zsc_v, [zidx + (b * 256 + j)], zb)
        for b in range(_B):
            acc = zsc_v[pl.ds(b * 256, 16)]
            for l in range(1, 16):
                acc = acc + zsc_v[pl.ds(b * 256 + l * 16, 16)]
            obufs[b][pl.ds(ch * _CH, _CH)] = acc
        return carry

    lax.fori_loop(0, nch, chunk_body, 0)

    @pl.when(w < _NW - 1)
    def _full():
        for b in range(_B):
            pltpu.sync_copy(obufs[b], outs[b].at[pl.ds(start, _PW)])

    @pl.when(w == _NW - 1)
    def _tail():
        ntail = _N - (_NW - 1) * _PW
        for b in range(_B):
            pltpu.sync_copy(obufs[b].at[pl.ds(0, ntail)],
                            outs[b].at[pl.ds(start, ntail)])


_sc_call = functools.partial(
    pl.kernel,
    mesh=plsc.VectorSubcoreMesh(core_axis_name="c", subcore_axis_name="s"),
    out_type=[jax.ShapeDtypeStruct((_N,), jnp.float32) for _ in range(_B)],
    compiler_params=pltpu.CompilerParams(
        needs_layout_passes=False, use_tc_tiling_on_sc=False),
    scratch_types=[
        pltpu.VMEM((_IDX,), jnp.int32),
        pltpu.VMEM((_IDX, _NLAT), jnp.float32),
        pltpu.VMEM((_IDX,), jnp.float32),
        pltpu.VMEM((_CH,), jnp.int32),
        pltpu.VMEM((_B * _NLAT,), jnp.float32),
        pltpu.VMEM((_B * _M * _NLAT,), jnp.float32),
        pltpu.VMEM((_B * _NLAT * _CH,), jnp.float32),
        pltpu.VMEM((_PW,), jnp.float32),
        pltpu.VMEM((_PW,), jnp.float32),
        pltpu.VMEM((_PW,), jnp.float32),
        pltpu.VMEM((_PW,), jnp.float32),
        pltpu.SemaphoreType.DMA,
    ],
)(_sc_body)


def kernel(x, W_enc, b_enc, W_dec, W_bw, b_bw, neighbour_distance,
           neighbour_id, clustering_labels):
    wbwp = W_bw.reshape(_NLAT, _M, _NLAT).transpose(1, 0, 2).reshape(_M * _NLAT, _NLAT)
    bbwp = b_bw.reshape(_NLAT, _M).T.reshape(1, _M * _NLAT)
    wd_t = W_dec.T

    enc, inv2 = pl.pallas_call(
        _tc_body,
        out_shape=[
            jax.ShapeDtypeStruct((_B, _NLAT), jnp.float32),
            jax.ShapeDtypeStruct((_B, _M * _NLAT), jnp.float32),
        ],
    )(x, W_enc, b_enc.reshape(1, _NLAT), wbwp, bbwp)

    outs = _sc_call(
        wd_t,
        neighbour_id.reshape(-1),
        neighbour_distance.reshape(-1),
        clustering_labels,
        enc.reshape(-1),
        inv2.reshape(-1),
    )
    return jnp.stack(outs, axis=0)


# speedup vs baseline: 12.9807x; 1.5719x over previous
import functools

import jax
import jax.numpy as jnp
from jax import lax
from jax.experimental import pallas as pl
from jax.experimental.pallas import tpu as pltpu
from jax.experimental.pallas import tpu_sc as plsc

_N = 50000
_NLAT = 16
_M = 16
_MU = 10.0
_K = 16
_B = 4
_LO = 2.0 / 60.0 / _MU
_HI = 1.0 / 60.0

_NW = 32
_PW = 1568
_CH = 16
_IDX = _CH * _K
_NCH = _PW // _CH


def _tc_body(x_ref, we_ref, be_ref, wbwp_ref, bbwp_ref, enc_ref, inv2_ref):
    x = x_ref[...]
    we = we_ref[...]
    enc = lax.dot_general(
        x, we, (((1,), (1,)), ((), ())),
        preferred_element_type=jnp.float32,
        precision=lax.Precision.HIGHEST,
    ) + be_ref[...]
    enc_ref[...] = enc
    z = lax.dot_general(
        enc, wbwp_ref[...], (((1,), (1,)), ((), ())),
        preferred_element_type=jnp.float32,
        precision=lax.Precision.HIGHEST,
    ) + bbwp_ref[...]
    bw = (_HI - _LO) * jax.nn.sigmoid(z) + _LO
    inv2_ref[...] = 1.0 / (bw * _MU) ** 2


def _sc_body(wd_hbm, nid_hbm, d_hbm, lab_hbm, enc_hbm, inv2_hbm,
             out0, out1, out2, out3,
             nid_v, d_all, lab_all, rows_v, enc_v, inv2_v, zsc_v,
             ob0, ob1, ob2, ob3, sem0, sem1):
    obufs = (ob0, ob1, ob2, ob3)
    outs = (out0, out1, out2, out3)
    w = lax.axis_index("s") * 2 + lax.axis_index("c")
    start = jnp.minimum(w * _PW, _N - _PW)

    pltpu.sync_copy(enc_hbm, enc_v)
    pltpu.sync_copy(inv2_hbm, inv2_v)
    pltpu.sync_copy(lab_hbm.at[pl.ds(start, _PW)], lab_all)
    pltpu.sync_copy(nid_hbm.at[pl.ds(start * _K, _PW * _K)], nid_v)
    pltpu.sync_copy(d_hbm.at[pl.ds(start * _K, _PW * _K)], d_all)
    zidx = lax.iota(jnp.int32, 16) * 16

    def issue(ch, dst_off, sem):
        io = ch * _IDX
        pltpu.async_copy(wd_hbm.at[nid_v.at[pl.ds(io, 128)]],
                         rows_v.at[pl.ds(dst_off, 128)], sem)
        pltpu.async_copy(wd_hbm.at[nid_v.at[pl.ds(io + 128, 128)]],
                         rows_v.at[pl.ds(dst_off + 128, 128)], sem)

    def drain(dst_off, sem):
        pltpu.make_async_copy(wd_hbm.at[nid_v.at[pl.ds(0, 128)]],
                              rows_v.at[pl.ds(dst_off, 128)], sem).wait()
        pltpu.make_async_copy(wd_hbm.at[nid_v.at[pl.ds(0, 128)]],
                              rows_v.at[pl.ds(dst_off + 128, 128)], sem).wait()

    def compute(ch, po):
        labv = lab_all[pl.ds(ch * _CH, _CH)]
        for j in range(_CH):
            dv = d_all[pl.ds(ch * _IDX + j * _K, _K)]
            d2v = dv * dv
            s2 = jnp.float32(0.0)
            sg = jnp.zeros((16,), jnp.float32)
            sdg = jnp.zeros((16,), jnp.float32)
            for k in range(_K):
                g = rows_v[po + j * _K + k]
                d2k = d2v[k]
                s2 = s2 + d2k
                sg = sg + g
                sdg = sdg + jnp.full((16,), d2k) * g
            c = labv[j]
            s2v = jnp.full((16,), s2)
            for b in range(_B):
                iv = inv2_v[pl.ds(b * 256 + c * 16, 16)]
                num = sg - iv * sdg
                den = float(_K) - iv * s2v
                zb = enc_v[pl.ds(b * 16, 16)] * (num / den)
                plsc.store_scatter(zsc_v, [zidx + (b * 256 + j)], zb)
        for b in range(_B):
            acc = zsc_v[pl.ds(b * 256, 16)]
            for l in range(1, 16):
                acc = acc + zsc_v[pl.ds(b * 256 + l * 16, 16)]
            obufs[b][pl.ds(ch * _CH, _CH)] = acc

    issue(0, 0, sem0)

    def chunk_body(ch, carry):
        po = (ch % 2) * _IDX
        pn = _IDX - po
        even = (ch % 2) == 0
        not_last = ch < _NCH - 1

        @pl.when(jnp.logical_and(even, not_last))
        def _issue_odd():
            issue(ch + 1, pn, sem1)

        @pl.when(jnp.logical_and(jnp.logical_not(even), not_last))
        def _issue_even():
            issue(ch + 1, pn, sem0)

        @pl.when(even)
        def _drain_even():
            drain(po, sem0)

        @pl.when(jnp.logical_not(even))
        def _drain_odd():
            drain(po, sem1)

        compute(ch, po)
        return carry

    lax.fori_loop(0, _NCH, chunk_body, 0)

    for b in range(_B):
        pltpu.sync_copy(obufs[b], outs[b].at[pl.ds(start, _PW)])


_sc_call = functools.partial(
    pl.kernel,
    mesh=plsc.VectorSubcoreMesh(core_axis_name="c", subcore_axis_name="s"),
    out_type=[jax.ShapeDtypeStruct((_N,), jnp.float32) for _ in range(_B)],
    compiler_params=pltpu.CompilerParams(
        needs_layout_passes=False, use_tc_tiling_on_sc=False),
    scratch_types=[
        pltpu.VMEM((_PW * _K,), jnp.int32),
        pltpu.VMEM((_PW * _K,), jnp.float32),
        pltpu.VMEM((_PW,), jnp.int32),
        pltpu.VMEM((2 * _IDX, _NLAT), jnp.float32),
        pltpu.VMEM((_B * _NLAT,), jnp.float32),
        pltpu.VMEM((_B * _M * _NLAT,), jnp.float32),
        pltpu.VMEM((_B * _NLAT * _CH,), jnp.float32),
        pltpu.VMEM((_PW,), jnp.float32),
        pltpu.VMEM((_PW,), jnp.float32),
        pltpu.VMEM((_PW,), jnp.float32),
        pltpu.VMEM((_PW,), jnp.float32),
        pltpu.SemaphoreType.DMA,
        pltpu.SemaphoreType.DMA,
    ],
)(_sc_body)


def kernel(x, W_enc, b_enc, W_dec, W_bw, b_bw, neighbour_distance,
           neighbour_id, clustering_labels):
    wbwp = W_bw.reshape(_NLAT, _M, _NLAT).transpose(1, 0, 2).reshape(_M * _NLAT, _NLAT)
    bbwp = b_bw.reshape(_NLAT, _M).T.reshape(1, _M * _NLAT)
    wd_t = W_dec.T

    enc, inv2 = pl.pallas_call(
        _tc_body,
        out_shape=[
            jax.ShapeDtypeStruct((_B, _NLAT), jnp.float32),
            jax.ShapeDtypeStruct((_B, _M * _NLAT), jnp.float32),
        ],
    )(x, W_enc, b_enc.reshape(1, _NLAT), wbwp, bbwp)

    outs = _sc_call(
        wd_t,
        neighbour_id.reshape(-1),
        neighbour_distance.reshape(-1),
        clustering_labels,
        enc.reshape(-1),
        inv2.reshape(-1),
    )
    return jnp.stack(outs, axis=0)
